# baseline (speedup 1.0000x reference)
import functools

import jax
import jax.numpy as jnp
from jax import lax
from jax.experimental import pallas as pl
from jax.experimental.pallas import tpu as pltpu
from jax.experimental.pallas import tpu_sc as plsc

TOTAL = 16384
D_MODEL = 2048
NUM_CHUNKS = 64
CHUNK = TOTAL // NUM_CHUNKS
NSEL = 8
OUT_ROWS = NSEL * CHUNK
NC, NS, L = 2, 16, 16
NW = NC * NS
ROWS_PER_W = OUT_ROWS // NW
STAGE = 16
NSTAGES = ROWS_PER_W // STAGE
NBUF = 3
NVEC = ROWS_PER_W // L


def _cumsum8(vec, iota):
    acc = jnp.zeros((L,), jnp.int32)
    for s in range(NSEL):
        acc = acc + jnp.where(iota >= s, vec[s], 0)
    return acc


def _body(x_hbm, meta_hbm, out_hbm,
          meta_v, starts_v, adj_v, idx_v, bufs, sems_in, sems_out):
    wid = lax.axis_index("s") * NC + lax.axis_index("c")
    base = wid * ROWS_PER_W
    iota = lax.iota(jnp.int32, L)

    pltpu.sync_copy(meta_hbm, meta_v)
    seql = meta_v[pl.ds(0, L)]
    perm = jnp.where(iota < NSEL, meta_v[pl.ds(NSEL, L)], 0)
    starts = _cumsum8(seql, iota) - seql
    starts_v[...] = starts
    slp = plsc.load_gather(meta_v, [perm])
    ends = _cumsum8(slp, iota)
    adj_v[...] = plsc.load_gather(starts_v, [perm]) - (ends - slp)

    firsts = []
    for v in range(NVEC):
        t_out = base + (v * L) + iota
        c = lax.shift_right_logical(t_out, 8)
        within = jnp.bitwise_and(t_out, CHUNK - 1)
        selc = plsc.load_gather(meta_v, [c + 2 * NSEL])
        t = lax.shift_left(selc, 8) + within
        j = jnp.zeros((L,), jnp.int32)
        for s in range(NSEL):
            j += jnp.where(t >= ends[s], 1, 0).astype(jnp.int32)
        src = t + plsc.load_gather(adj_v, [j])
        for k in range(L // STAGE):
            firsts.append(pl.multiple_of(src[k * STAGE], STAGE))

    cps = []
    for s in range(NSTAGES):
        cps.append(pltpu.async_copy(
            x_hbm.at[pl.ds(firsts[s], STAGE)],
            out_hbm.at[pl.ds(base + s * STAGE, STAGE)],
            sems_in[s % NBUF] if s < NBUF else sems_out[s - NBUF]))
    for cp in cps:
        cp.wait()


def _flat_body(x_hbm, meta_hbm, out_hbm,
               meta_v, starts_v, adj_v, idx_v, *rest):
    bufs = rest[:NBUF]
    sems_in = rest[NBUF:2 * NBUF]
    sems_out = rest[2 * NBUF:]
    _body(x_hbm, meta_hbm, out_hbm,
          meta_v, starts_v, adj_v, idx_v, bufs, sems_in, sems_out)


@jax.jit
def _dispatch(x_global, meta):
    mesh = plsc.VectorSubcoreMesh(core_axis_name="c", subcore_axis_name="s")
    run = functools.partial(
        pl.kernel,
        mesh=mesh,
        compiler_params=pltpu.CompilerParams(needs_layout_passes=False),
        out_type=jax.ShapeDtypeStruct((OUT_ROWS, D_MODEL), jnp.float32),
        scratch_types=[
            pltpu.VMEM((2 * L,), jnp.int32),
            pltpu.VMEM((L,), jnp.int32),
            pltpu.VMEM((L,), jnp.int32),
            pltpu.VMEM((ROWS_PER_W,), jnp.int32),
        ]
        + [pltpu.VMEM((STAGE, D_MODEL), jnp.float32)] * NBUF
        + [pltpu.SemaphoreType.DMA] * (2 * NBUF),
    )(_flat_body)
    return run(x_global, meta)


def kernel(x_global, seqlens, seqlens_perm_idxs, chunk_sel):
    meta = jnp.concatenate([
        jnp.asarray(seqlens, jnp.int32),
        jnp.asarray(seqlens_perm_idxs, jnp.int32),
        jnp.asarray(chunk_sel, jnp.int32),
        jnp.zeros((NSEL,), jnp.int32),
    ])
    return _dispatch(x_global, meta)


# speedup vs baseline: 16.3520x; 16.3520x over previous
import functools

import jax
import jax.numpy as jnp
from jax import lax
from jax.experimental import pallas as pl
from jax.experimental.pallas import tpu as pltpu
from jax.experimental.pallas import tpu_sc as plsc

TOTAL = 16384
D_MODEL = 2048
NUM_CHUNKS = 64
CHUNK = TOTAL // NUM_CHUNKS
NSEL = 8
OUT_ROWS = NSEL * CHUNK
NC, NS, L = 2, 16, 16
NW = NC * NS
ROWS_PER_W = OUT_ROWS // NW
STAGE = 16
NSTAGES = ROWS_PER_W // STAGE
NBUF = 3
NVEC = ROWS_PER_W // L


def _cumsum8(vec, iota):
    acc = jnp.zeros((L,), jnp.int32)
    for s in range(NSEL):
        acc = acc + jnp.where(iota >= s, vec[s], 0)
    return acc


def _body(x_hbm, meta_hbm, out_hbm,
          meta_v, starts_v, adj_v, bufs, sems_in, sems_out):
    wid = lax.axis_index("s") * NC + lax.axis_index("c")
    base = wid * ROWS_PER_W
    iota = lax.iota(jnp.int32, L)

    pltpu.sync_copy(meta_hbm, meta_v)
    seql = meta_v[pl.ds(0, L)]
    perm = jnp.where(iota < NSEL, meta_v[pl.ds(NSEL, L)], 0)
    starts = _cumsum8(seql, iota) - seql
    starts_v[...] = starts
    slp = plsc.load_gather(meta_v, [perm])
    ends = _cumsum8(slp, iota)
    adj_v[...] = plsc.load_gather(starts_v, [perm]) - (ends - slp)

    firsts = []
    for v in range(NVEC):
        t_out = base + (v * L) + iota
        c = lax.shift_right_logical(t_out, 8)
        within = jnp.bitwise_and(t_out, CHUNK - 1)
        selc = plsc.load_gather(meta_v, [c + 2 * NSEL])
        t = lax.shift_left(selc, 8) + within
        j = jnp.zeros((L,), jnp.int32)
        for s in range(NSEL):
            j += jnp.where(t >= ends[s], 1, 0).astype(jnp.int32)
        src = t + plsc.load_gather(adj_v, [j])
        for k in range(L // STAGE):
            firsts.append(pl.multiple_of(src[k * STAGE], STAGE))

    cp_in = [None] * NSTAGES
    cp_out = [None] * NSTAGES
    for s in range(min(NBUF, NSTAGES)):
        cp_in[s] = pltpu.async_copy(
            x_hbm.at[pl.ds(firsts[s], STAGE)], bufs[s], sems_in[s])
    out_waited = [False] * NSTAGES
    for s in range(NSTAGES):
        b = s % NBUF
        cp_in[s].wait()
        cp_out[s] = pltpu.async_copy(
            bufs[b], out_hbm.at[pl.ds(base + s * STAGE, STAGE)], sems_out[b])
        nxt = s + NBUF
        if nxt < NSTAGES:
            cp_out[s].wait()
            out_waited[s] = True
            cp_in[nxt] = pltpu.async_copy(
                x_hbm.at[pl.ds(firsts[nxt], STAGE)], bufs[b], sems_in[b])
    for s in range(NSTAGES):
        if not out_waited[s]:
            cp_out[s].wait()


def _flat_body(x_hbm, meta_hbm, out_hbm,
               meta_v, starts_v, adj_v, *rest):
    bufs = rest[:NBUF]
    sems_in = rest[NBUF:2 * NBUF]
    sems_out = rest[2 * NBUF:]
    _body(x_hbm, meta_hbm, out_hbm,
          meta_v, starts_v, adj_v, bufs, sems_in, sems_out)


@jax.jit
def _dispatch(x_global, meta):
    mesh = plsc.VectorSubcoreMesh(core_axis_name="c", subcore_axis_name="s")
    run = functools.partial(
        pl.kernel,
        mesh=mesh,
        compiler_params=pltpu.CompilerParams(needs_layout_passes=False),
        out_type=jax.ShapeDtypeStruct((OUT_ROWS, D_MODEL), jnp.float32),
        scratch_types=[
            pltpu.VMEM((2 * L,), jnp.int32),
            pltpu.VMEM((L,), jnp.int32),
            pltpu.VMEM((L,), jnp.int32),
        ]
        + [pltpu.VMEM((STAGE, D_MODEL), jnp.float32)] * NBUF
        + [pltpu.SemaphoreType.DMA] * (2 * NBUF),
    )(_flat_body)
    return run(x_global, meta)


def kernel(x_global, seqlens, seqlens_perm_idxs, chunk_sel):
    meta = jnp.concatenate([
        jnp.asarray(seqlens, jnp.int32),
        jnp.asarray(seqlens_perm_idxs, jnp.int32),
        jnp.asarray(chunk_sel, jnp.int32),
        jnp.zeros((NSEL,), jnp.int32),
    ])
    return _dispatch(x_global, meta)


# speedup vs baseline: 16.3694x; 1.0011x over previous
import functools

import jax
import jax.numpy as jnp
from jax import lax
from jax.experimental import pallas as pl
from jax.experimental.pallas import tpu as pltpu
from jax.experimental.pallas import tpu_sc as plsc

TOTAL = 16384
D_MODEL = 2048
NUM_CHUNKS = 64
CHUNK = TOTAL // NUM_CHUNKS
NSEL = 8
OUT_ROWS = NSEL * CHUNK
NC, NS, L = 2, 16, 16
NW = NC * NS
ROWS_PER_W = OUT_ROWS // NW
STAGES = ((0, 8), (8, 16), (24, 16), (40, 16), (56, 8))
NBUF = 4
NVEC = ROWS_PER_W // L


def _cumsum8(vec, iota):
    acc = jnp.zeros((L,), jnp.int32)
    for s in range(NSEL):
        acc = acc + jnp.where(iota >= s, vec[s], 0)
    return acc


def _body(x_hbm, meta_hbm, out_hbm,
          meta_v, starts_v, adj_v, bufs, sems_in, sems_out):
    wid = lax.axis_index("s") * NC + lax.axis_index("c")
    base = wid * ROWS_PER_W
    iota = lax.iota(jnp.int32, L)

    pltpu.sync_copy(meta_hbm, meta_v)
    seql = meta_v[pl.ds(0, L)]
    perm = jnp.where(iota < NSEL, meta_v[pl.ds(NSEL, L)], 0)
    starts = _cumsum8(seql, iota) - seql
    starts_v[...] = starts
    slp = plsc.load_gather(meta_v, [perm])
    ends = _cumsum8(slp, iota)
    adj_v[...] = plsc.load_gather(starts_v, [perm]) - (ends - slp)

    srcs = []
    for v in range(NVEC):
        t_out = base + (v * L) + iota
        c = lax.shift_right_logical(t_out, 8)
        within = jnp.bitwise_and(t_out, CHUNK - 1)
        selc = plsc.load_gather(meta_v, [c + 2 * NSEL])
        t = lax.shift_left(selc, 8) + within
        j = jnp.zeros((L,), jnp.int32)
        for s in range(NSEL):
            j += jnp.where(t >= ends[s], 1, 0).astype(jnp.int32)
        srcs.append(t + plsc.load_gather(adj_v, [j]))
    firsts = [pl.multiple_of(srcs[off // L][off % L], 8) for off, _ in STAGES]

    nst = len(STAGES)
    cp_in = [None] * nst
    cp_out = [None] * nst
    for s in range(NBUF):
        cp_in[s] = pltpu.async_copy(
            x_hbm.at[pl.ds(firsts[s], STAGES[s][1])], bufs[s], sems_in[s])
    for s in range(nst):
        b = s % NBUF
        off, n = STAGES[s]
        cp_in[s].wait()
        cp_out[s] = pltpu.async_copy(
            bufs[b], out_hbm.at[pl.ds(base + off, n)], sems_out[b])
        if s == 0 and nst > NBUF:
            cp_out[0].wait()
            cp_in[NBUF] = pltpu.async_copy(
                x_hbm.at[pl.ds(firsts[NBUF], STAGES[NBUF][1])],
                bufs[0], sems_in[0])
    for s in range(1, nst):
        cp_out[s].wait()


def _flat_body(x_hbm, meta_hbm, out_hbm,
               meta_v, starts_v, adj_v, *rest):
    bufs = rest[:NBUF]
    sems_in = rest[NBUF:2 * NBUF]
    sems_out = rest[2 * NBUF:]
    _body(x_hbm, meta_hbm, out_hbm,
          meta_v, starts_v, adj_v, bufs, sems_in, sems_out)


@jax.jit
def _dispatch(x_global, meta):
    mesh = plsc.VectorSubcoreMesh(core_axis_name="c", subcore_axis_name="s")
    run = functools.partial(
        pl.kernel,
        mesh=mesh,
        compiler_params=pltpu.CompilerParams(needs_layout_passes=False),
        out_type=jax.ShapeDtypeStruct((OUT_ROWS, D_MODEL), jnp.float32),
        scratch_types=[
            pltpu.VMEM((2 * L,), jnp.int32),
            pltpu.VMEM((L,), jnp.int32),
            pltpu.VMEM((L,), jnp.int32),
        ]
        + [pltpu.VMEM((STAGES[s][1], D_MODEL), jnp.float32)
           for s in range(NBUF)]
        + [pltpu.SemaphoreType.DMA] * (2 * NBUF),
    )(_flat_body)
    return run(x_global, meta)


def kernel(x_global, seqlens, seqlens_perm_idxs, chunk_sel):
    meta = jnp.concatenate([
        jnp.asarray(seqlens, jnp.int32),
        jnp.asarray(seqlens_perm_idxs, jnp.int32),
        jnp.asarray(chunk_sel, jnp.int32),
        jnp.zeros((NSEL,), jnp.int32),
    ])
    return _dispatch(x_global, meta)
